# baseline (speedup 1.0000x reference)
import functools

import jax
import jax.numpy as jnp
from jax import lax
from jax.experimental import pallas as pl
from jax.experimental.pallas import tpu as pltpu
from jax.experimental.pallas import tpu_sc as plsc

E_TOT, NM_TOT, NG_TOT, D_F, H_F = 320000, 10000, 10000, 128, 128
NC, NS = 2, 16
NW = NC * NS
NSL = 2
E_SL = E_TOT // NSL
CH = 40
NSET = 5
NSETS = 5
NZ = NG_TOT // CH
BLK = 2000


def _pre_body(mesh_ref, grid_ref, w1s_ref, w1d_ref, b1_ref, ps_ref, pd_ref):
    ps_ref[...] = jnp.dot(mesh_ref[...], w1s_ref[...],
                          preferred_element_type=jnp.float32)
    pd_ref[...] = jnp.dot(grid_ref[...], w1d_ref[...],
                          preferred_element_type=jnp.float32) + b1_ref[...]


def _edge_body(ef_ref, g_ref, w1e_ref, w2_ref, b2_ref, s_ref, b_ref,
               out_ref):
    h = (jnp.dot(ef_ref[...], w1e_ref[...], preferred_element_type=jnp.float32)
         + g_ref[...])
    h = h * (1.0 / (1.0 + jnp.exp(-h)))
    h = jnp.dot(h, w2_ref[...], preferred_element_type=jnp.float32) + b2_ref[...]
    mu = jnp.mean(h, axis=-1, keepdims=True)
    var = jnp.mean((h - mu) ** 2, axis=-1, keepdims=True)
    out_ref[...] = (h - mu) * lax.rsqrt(var + 1e-5) * s_ref[...] + b_ref[...]


def _node_body(part_ref, grid_ref, w1a_ref, w1b_ref, b1_ref, w2_ref, b2_ref,
               s_ref, b_ref, out_ref):
    agg = part_ref[0]
    for k in range(1, NSL * NC):
        agg = agg + part_ref[k]
    g = grid_ref[...]
    h = (jnp.dot(agg, w1a_ref[...], preferred_element_type=jnp.float32)
         + jnp.dot(g, w1b_ref[...], preferred_element_type=jnp.float32)
         + b1_ref[...])
    h = h * (1.0 / (1.0 + jnp.exp(-h)))
    h = jnp.dot(h, w2_ref[...], preferred_element_type=jnp.float32) + b2_ref[...]
    mu = jnp.mean(h, axis=-1, keepdims=True)
    var = jnp.mean((h - mu) ** 2, axis=-1, keepdims=True)
    out_ref[...] = (h - mu) * lax.rsqrt(var + 1e-5) * s_ref[...] + b_ref[...] + g


_sc_mesh = plsc.VectorSubcoreMesh(core_axis_name="c", subcore_axis_name="s")


def _make_gather(ne):
    epw = ne // NW
    niter = epw // CH // NSET

    @functools.partial(
        pl.kernel,
        out_type=jax.ShapeDtypeStruct((ne, H_F), jnp.float32),
        mesh=_sc_mesh,
        scratch_types=(
            [pltpu.VMEM((2 * epw,), jnp.int32)]
            + [pltpu.VMEM((2 * CH, H_F), jnp.float32) for _ in range(NSET)]
            + [pltpu.SemaphoreType.DMA for _ in range(2 * NSET)]
        ),
    )
    def gather_kernel(tbl_hbm, cidx_hbm, g_hbm, *scr):
        cidx_all = scr[0]
        rbuf = scr[1:1 + NSET]
        ga = scr[1 + NSET:1 + 2 * NSET]
        ws = scr[1 + 2 * NSET:1 + 3 * NSET]
        wid = lax.axis_index("s") * NC + lax.axis_index("c")
        pltpu.sync_copy(cidx_hbm.at[pl.ds(wid * 2 * epw, 2 * epw)], cidx_all)

        def body(j, carry):
            hs = []
            for b in range(NSET):
                loc = (j * NSET + b) * 2 * CH
                hs.append(pltpu.async_copy(
                    tbl_hbm.at[cidx_all.at[pl.ds(loc, 2 * CH)]], rbuf[b],
                    ga[b]))
            wh = []
            for b in range(NSET):
                base = wid * epw + (j * NSET + b) * CH
                hs[b].wait()

                def addrow(r, carry, _rb=rbuf[b]):
                    for c in range(H_F // 16):
                        sl = pl.ds(c * 16, 16)
                        _rb[r, sl] = _rb[2 * r, sl] + _rb[2 * r + 1, sl]
                    return carry

                lax.fori_loop(0, CH, addrow, 0)
                wh.append(pltpu.async_copy(rbuf[b].at[pl.ds(0, CH)],
                                           g_hbm.at[pl.ds(base, CH)], ws[b]))
            for h in wh:
                h.wait()
            return carry

        lax.fori_loop(0, niter, body, 0)

    return gather_kernel


def _make_scatter(ne):
    epw = ne // NW
    nchs = epw // CH

    @functools.partial(
        pl.kernel,
        out_type=jax.ShapeDtypeStruct((NC, NG_TOT, H_F), jnp.float32),
        mesh=_sc_mesh,
        scratch_types=(
            [pltpu.VMEM_SHARED((NG_TOT, H_F), jnp.float32)]
            + [pltpu.VMEM((CH, H_F), jnp.float32) for _ in range(NSETS)]
            + [pltpu.VMEM((CH,), jnp.int32) for _ in range(NSETS)]
            + [pltpu.SemaphoreType.DMA for _ in range(3 * NSETS)]
        ),
    )
    def scatter_kernel(oute_hbm, didx_hbm, part_hbm, *scr):
        table = scr[0]
        rv = scr[1:1 + NSETS]
        iv = scr[1 + NSETS:1 + 2 * NSETS]
        fs = scr[1 + 2 * NSETS:1 + 3 * NSETS]
        es = scr[1 + 3 * NSETS:1 + 4 * NSETS]
        ss = scr[1 + 4 * NSETS:1 + 5 * NSETS]
        cid = lax.axis_index("c")
        sid = lax.axis_index("s")
        wid = sid * NC + cid

        def zrow(k, carry):
            r = k // (H_F // 16)
            c = k % (H_F // 16)
            rv[0][r, pl.ds(c * 16, 16)] = jnp.zeros((16,), jnp.float32)
            return carry

        lax.fori_loop(0, CH * (H_F // 16), zrow, 0)

        def zcopy(t, carry):
            zi = sid + t * NS

            @pl.when(zi < NZ)
            def _():
                pltpu.sync_copy(rv[0], table.at[pl.ds(zi * CH, CH)])

            return carry

        lax.fori_loop(0, (NZ + NS - 1) // NS, zcopy, 0)
        plsc.subcore_barrier()

        def body(j, carry):
            fh = []
            for b in range(NSETS):
                base = wid * epw + (j * NSETS + b) * CH
                fh.append((
                    pltpu.async_copy(oute_hbm.at[pl.ds(base, CH)], rv[b],
                                     fs[b]),
                    pltpu.async_copy(didx_hbm.at[pl.ds(base, CH)], iv[b],
                                     es[b]),
                ))
            sh = []
            for b in range(NSETS):
                fh[b][0].wait()
                fh[b][1].wait()
                sh.append(pltpu.async_copy(rv[b], table.at[iv[b]], ss[b],
                                           add=True))
            for h in sh:
                h.wait()
            return carry

        lax.fori_loop(0, nchs // NSETS, body, 0)
        plsc.subcore_barrier()

        def wout(t, carry):
            wi = sid + t * NS

            @pl.when(wi < NZ)
            def _():
                pltpu.sync_copy(table.at[pl.ds(wi * CH, CH)], rv[0])
                pltpu.sync_copy(rv[0], part_hbm.at[cid, pl.ds(wi * CH, CH)])

            return carry

        lax.fori_loop(0, (NZ + NS - 1) // NS, wout, 0)

    return scatter_kernel


_gather_sl = _make_gather(E_SL)
_scatter_sl = _make_scatter(E_SL)


def kernel(m2g_efeat, grid_nfeat, mesh_nfeat, edge_index,
           e_W1, e_b1, e_W2, e_b2, e_ln_s, e_ln_b,
           n_W1, n_b1, n_W2, n_b2, n_ln_s, n_ln_b):
    src = edge_index[0]
    dst = edge_index[1]
    w1e, w1s, w1d = e_W1[:D_F], e_W1[D_F:2 * D_F], e_W1[2 * D_F:]
    n_w1a, n_w1b = n_W1[:D_F], n_W1[D_F:]
    eb1 = e_b1.reshape(1, H_F)
    eb2 = e_b2.reshape(1, D_F)
    es = e_ln_s.reshape(1, D_F)
    eb = e_ln_b.reshape(1, D_F)
    nb1 = n_b1.reshape(1, H_F)
    nb2 = n_b2.reshape(1, D_F)
    nss = n_ln_s.reshape(1, D_F)
    nbb = n_ln_b.reshape(1, D_F)

    row_spec = pl.BlockSpec((BLK, D_F), lambda i: (i, 0))
    full_w = pl.BlockSpec((D_F, H_F), lambda i: (0, 0))
    full_b = pl.BlockSpec((1, H_F), lambda i: (0, 0))

    pre_src, pre_dst = pl.pallas_call(
        _pre_body,
        grid=(NM_TOT // BLK,),
        in_specs=[row_spec, row_spec, full_w, full_w, full_b],
        out_specs=[row_spec, row_spec],
        out_shape=[jax.ShapeDtypeStruct((NM_TOT, H_F), jnp.float32),
                   jax.ShapeDtypeStruct((NG_TOT, H_F), jnp.float32)],
    )(mesh_nfeat, grid_nfeat, w1s, w1d, eb1)

    tbl = jnp.concatenate([pre_src, pre_dst], axis=0)
    cidx = jnp.stack([src, dst + NM_TOT], axis=-1).reshape(-1)

    parts = []
    nblk_sl = E_SL // BLK
    for i in range(NSL):
        dst_i = lax.slice_in_dim(dst, i * E_SL, (i + 1) * E_SL)
        cidx_i = lax.slice_in_dim(cidx, 2 * i * E_SL, 2 * (i + 1) * E_SL)
        g_i = _gather_sl(tbl, cidx_i)

        ef_spec = pl.BlockSpec((BLK, D_F),
                               lambda j, off=i * nblk_sl: (off + j, 0))
        out_e = pl.pallas_call(
            _edge_body,
            grid=(nblk_sl,),
            in_specs=[ef_spec, row_spec, full_w, full_w, full_b,
                      full_b, full_b],
            out_specs=row_spec,
            out_shape=jax.ShapeDtypeStruct((E_SL, H_F), jnp.float32),
        )(m2g_efeat, g_i, w1e, e_W2, eb2, es, eb)

        parts.append(_scatter_sl(out_e, dst_i))

    part = jnp.concatenate(parts, axis=0)

    out = pl.pallas_call(
        _node_body,
        grid=(NG_TOT // BLK,),
        in_specs=[pl.BlockSpec((NSL * NC, BLK, H_F), lambda i: (0, i, 0)),
                  row_spec, full_w, full_w, full_b, full_w, full_b,
                  full_b, full_b],
        out_specs=row_spec,
        out_shape=jax.ShapeDtypeStruct((NG_TOT, D_F), jnp.float32),
    )(part, grid_nfeat, n_w1a, n_w1b, nb1, n_W2, nb2, nss, nbb)

    return out


# speedup vs baseline: 1.9011x; 1.9011x over previous
import functools

import jax
import jax.numpy as jnp
from jax import lax
from jax.experimental import pallas as pl
from jax.experimental.pallas import tpu as pltpu
from jax.experimental.pallas import tpu_sc as plsc

E_TOT, NM_TOT, NG_TOT, D_F, H_F = 320000, 10000, 10000, 128, 128
NC, NS = 2, 16
NW = NC * NS
NSL = 2
E_SL = E_TOT // NSL
CH = 40
NSET = 5
NSETS = 5
NZ = NG_TOT // CH
BLK = 2000


def _pre_body(mesh_ref, grid_ref, w1s_ref, w1d_ref, b1_ref, ps_ref, pd_ref):
    ps_ref[...] = jnp.dot(mesh_ref[...], w1s_ref[...],
                          preferred_element_type=jnp.float32)
    pd_ref[...] = jnp.dot(grid_ref[...], w1d_ref[...],
                          preferred_element_type=jnp.float32) + b1_ref[...]


def _edge_body(ef_ref, g_ref, w1e_ref, w2_ref, b2_ref, s_ref, b_ref,
               out_ref):
    h = (jnp.dot(ef_ref[...], w1e_ref[...], preferred_element_type=jnp.float32)
         + g_ref[...])
    h = h * (1.0 / (1.0 + jnp.exp(-h)))
    h = jnp.dot(h, w2_ref[...], preferred_element_type=jnp.float32) + b2_ref[...]
    mu = jnp.mean(h, axis=-1, keepdims=True)
    var = jnp.mean((h - mu) ** 2, axis=-1, keepdims=True)
    out_ref[...] = (h - mu) * lax.rsqrt(var + 1e-5) * s_ref[...] + b_ref[...]


def _node_body(part_ref, grid_ref, w1a_ref, w1b_ref, b1_ref, w2_ref, b2_ref,
               s_ref, b_ref, out_ref):
    agg = part_ref[0]
    for k in range(1, NSL * NC):
        agg = agg + part_ref[k]
    g = grid_ref[...]
    h = (jnp.dot(agg, w1a_ref[...], preferred_element_type=jnp.float32)
         + jnp.dot(g, w1b_ref[...], preferred_element_type=jnp.float32)
         + b1_ref[...])
    h = h * (1.0 / (1.0 + jnp.exp(-h)))
    h = jnp.dot(h, w2_ref[...], preferred_element_type=jnp.float32) + b2_ref[...]
    mu = jnp.mean(h, axis=-1, keepdims=True)
    var = jnp.mean((h - mu) ** 2, axis=-1, keepdims=True)
    out_ref[...] = (h - mu) * lax.rsqrt(var + 1e-5) * s_ref[...] + b_ref[...] + g


_sc_mesh = plsc.VectorSubcoreMesh(core_axis_name="c", subcore_axis_name="s")


def _make_gather(ne):
    epw = ne // NW
    niter = epw // CH // NSET

    @functools.partial(
        pl.kernel,
        out_type=jax.ShapeDtypeStruct((ne, H_F), jnp.float32),
        mesh=_sc_mesh,
        scratch_types=(
            [pltpu.VMEM((epw,), jnp.int32),
             pltpu.VMEM((epw,), jnp.int32)]
            + [pltpu.VMEM((CH, H_F), jnp.float32) for _ in range(2 * NSET)]
            + [pltpu.SemaphoreType.DMA for _ in range(3 * NSET)]
        ),
    )
    def gather_kernel(ps_hbm, pd_hbm, sidx_hbm, didx_hbm, g_hbm, *scr):
        sidx_all, didx_all = scr[0], scr[1]
        ra = scr[2:2 + NSET]
        rb = scr[2 + NSET:2 + 2 * NSET]
        ga = scr[2 + 2 * NSET:2 + 3 * NSET]
        gb = scr[2 + 3 * NSET:2 + 4 * NSET]
        ws = scr[2 + 4 * NSET:2 + 5 * NSET]
        wid = lax.axis_index("s") * NC + lax.axis_index("c")
        pltpu.sync_copy(sidx_hbm.at[pl.ds(wid * epw, epw)], sidx_all)
        pltpu.sync_copy(didx_hbm.at[pl.ds(wid * epw, epw)], didx_all)

        def body(j, carry):
            hs = []
            for b in range(NSET):
                loc = (j * NSET + b) * CH
                hs.append((
                    pltpu.async_copy(
                        ps_hbm.at[sidx_all.at[pl.ds(loc, CH)]], ra[b], ga[b]),
                    pltpu.async_copy(
                        pd_hbm.at[didx_all.at[pl.ds(loc, CH)]], rb[b], gb[b]),
                ))
            wh = []
            for b in range(NSET):
                base = wid * epw + (j * NSET + b) * CH
                hs[b][0].wait()
                hs[b][1].wait()

                def addrow(q, carry, _ra=ra[b], _rb=rb[b]):
                    for u in range(4):
                        r = q * 4 + u
                        for c in range(H_F // 16):
                            sl = pl.ds(c * 16, 16)
                            _rb[r, sl] = _rb[r, sl] + _ra[r, sl]
                    return carry

                lax.fori_loop(0, CH // 4, addrow, 0)
                wh.append(pltpu.async_copy(rb[b], g_hbm.at[pl.ds(base, CH)],
                                           ws[b]))
            for h in wh:
                h.wait()
            return carry

        lax.fori_loop(0, niter, body, 0)

    return gather_kernel


def _make_scatter(ne):
    epw = ne // NW
    nchs = epw // CH

    @functools.partial(
        pl.kernel,
        out_type=jax.ShapeDtypeStruct((NC, NG_TOT, H_F), jnp.float32),
        mesh=_sc_mesh,
        scratch_types=(
            [pltpu.VMEM_SHARED((NG_TOT, H_F), jnp.float32)]
            + [pltpu.VMEM((CH, H_F), jnp.float32) for _ in range(NSETS)]
            + [pltpu.VMEM((CH,), jnp.int32) for _ in range(NSETS)]
            + [pltpu.SemaphoreType.DMA for _ in range(3 * NSETS)]
        ),
    )
    def scatter_kernel(oute_hbm, didx_hbm, part_hbm, *scr):
        table = scr[0]
        rv = scr[1:1 + NSETS]
        iv = scr[1 + NSETS:1 + 2 * NSETS]
        fs = scr[1 + 2 * NSETS:1 + 3 * NSETS]
        es = scr[1 + 3 * NSETS:1 + 4 * NSETS]
        ss = scr[1 + 4 * NSETS:1 + 5 * NSETS]
        cid = lax.axis_index("c")
        sid = lax.axis_index("s")
        wid = sid * NC + cid

        def zrow(k, carry):
            r = k // (H_F // 16)
            c = k % (H_F // 16)
            rv[0][r, pl.ds(c * 16, 16)] = jnp.zeros((16,), jnp.float32)
            return carry

        lax.fori_loop(0, CH * (H_F // 16), zrow, 0)

        def zcopy(t, carry):
            zi = sid + t * NS

            @pl.when(zi < NZ)
            def _():
                pltpu.sync_copy(rv[0], table.at[pl.ds(zi * CH, CH)])

            return carry

        lax.fori_loop(0, (NZ + NS - 1) // NS, zcopy, 0)
        plsc.subcore_barrier()

        def body(j, carry):
            fh = []
            for b in range(NSETS):
                base = wid * epw + (j * NSETS + b) * CH
                fh.append((
                    pltpu.async_copy(oute_hbm.at[pl.ds(base, CH)], rv[b],
                                     fs[b]),
                    pltpu.async_copy(didx_hbm.at[pl.ds(base, CH)], iv[b],
                                     es[b]),
                ))
            sh = []
            for b in range(NSETS):
                fh[b][0].wait()
                fh[b][1].wait()
                sh.append(pltpu.async_copy(rv[b], table.at[iv[b]], ss[b],
                                           add=True))
            for h in sh:
                h.wait()
            return carry

        lax.fori_loop(0, nchs // NSETS, body, 0)
        plsc.subcore_barrier()

        def wout(t, carry):
            wi = sid + t * NS

            @pl.when(wi < NZ)
            def _():
                pltpu.sync_copy(table.at[pl.ds(wi * CH, CH)], rv[0])
                pltpu.sync_copy(rv[0], part_hbm.at[cid, pl.ds(wi * CH, CH)])

            return carry

        lax.fori_loop(0, (NZ + NS - 1) // NS, wout, 0)

    return scatter_kernel


_gather_sl = _make_gather(E_SL)
_scatter_sl = _make_scatter(E_SL)


def kernel(m2g_efeat, grid_nfeat, mesh_nfeat, edge_index,
           e_W1, e_b1, e_W2, e_b2, e_ln_s, e_ln_b,
           n_W1, n_b1, n_W2, n_b2, n_ln_s, n_ln_b):
    src = edge_index[0]
    dst = edge_index[1]
    w1e, w1s, w1d = e_W1[:D_F], e_W1[D_F:2 * D_F], e_W1[2 * D_F:]
    n_w1a, n_w1b = n_W1[:D_F], n_W1[D_F:]
    eb1 = e_b1.reshape(1, H_F)
    eb2 = e_b2.reshape(1, D_F)
    es = e_ln_s.reshape(1, D_F)
    eb = e_ln_b.reshape(1, D_F)
    nb1 = n_b1.reshape(1, H_F)
    nb2 = n_b2.reshape(1, D_F)
    nss = n_ln_s.reshape(1, D_F)
    nbb = n_ln_b.reshape(1, D_F)

    row_spec = pl.BlockSpec((BLK, D_F), lambda i: (i, 0))
    full_w = pl.BlockSpec((D_F, H_F), lambda i: (0, 0))
    full_b = pl.BlockSpec((1, H_F), lambda i: (0, 0))

    pre_src, pre_dst = pl.pallas_call(
        _pre_body,
        grid=(NM_TOT // BLK,),
        in_specs=[row_spec, row_spec, full_w, full_w, full_b],
        out_specs=[row_spec, row_spec],
        out_shape=[jax.ShapeDtypeStruct((NM_TOT, H_F), jnp.float32),
                   jax.ShapeDtypeStruct((NG_TOT, H_F), jnp.float32)],
    )(mesh_nfeat, grid_nfeat, w1s, w1d, eb1)

    parts = []
    nblk_sl = E_SL // BLK
    for i in range(NSL):
        src_i = lax.slice_in_dim(src, i * E_SL, (i + 1) * E_SL)
        dst_i = lax.slice_in_dim(dst, i * E_SL, (i + 1) * E_SL)
        g_i = _gather_sl(pre_src, pre_dst, src_i, dst_i)

        ef_spec = pl.BlockSpec((BLK, D_F),
                               lambda j, off=i * nblk_sl: (off + j, 0))
        out_e = pl.pallas_call(
            _edge_body,
            grid=(nblk_sl,),
            in_specs=[ef_spec, row_spec, full_w, full_w, full_b,
                      full_b, full_b],
            out_specs=row_spec,
            out_shape=jax.ShapeDtypeStruct((E_SL, H_F), jnp.float32),
        )(m2g_efeat, g_i, w1e, e_W2, eb2, es, eb)

        parts.append(_scatter_sl(out_e, dst_i))

    part = jnp.concatenate(parts, axis=0)

    out = pl.pallas_call(
        _node_body,
        grid=(NG_TOT // BLK,),
        in_specs=[pl.BlockSpec((NSL * NC, BLK, H_F), lambda i: (0, i, 0)),
                  row_spec, full_w, full_w, full_b, full_w, full_b,
                  full_b, full_b],
        out_specs=row_spec,
        out_shape=jax.ShapeDtypeStruct((NG_TOT, D_F), jnp.float32),
    )(part, grid_nfeat, n_w1a, n_w1b, nb1, n_W2, nb2, nss, nbb)

    return out
